# baseline (speedup 1.0000x reference)
import jax
import jax.numpy as jnp
from jax.experimental import pallas as pl
from jax.experimental.pallas import tpu as pltpu

NCHUNK = 16
PACK = 8


def _linear_kernel(x_hbm, w_ref, b_ref, o_ref, xbuf, wbig_ref, sems):
    R = x_hbm.shape[0]
    Kp = x_hbm.shape[1]
    N = w_ref.shape[0]
    K = w_ref.shape[1]
    ch = R // NCHUNK

    for i in range(NCHUNK):
        pltpu.make_async_copy(
            x_hbm.at[pl.ds(i * ch, ch), :],
            xbuf.at[pl.ds(i * ch, ch), :],
            sems.at[i],
        ).start()

    wbig_ref[...] = jnp.zeros((K, Kp), dtype=jnp.float32)
    for a in range(PACK):
        wbig_ref[pl.ds(a * N, N), pl.ds(a * K, K)] = w_ref[...]

    btile = jnp.concatenate([b_ref[...]] * PACK, axis=1)

    for i in range(NCHUNK):
        pltpu.make_async_copy(
            x_hbm.at[pl.ds(i * ch, ch), :],
            xbuf.at[pl.ds(i * ch, ch), :],
            sems.at[i],
        ).wait()
        acc = jax.lax.dot_general(
            xbuf[pl.ds(i * ch, ch), :], wbig_ref[...],
            dimension_numbers=(((1,), (1,)), ((), ())),
            preferred_element_type=jnp.float32,
        )
        o_ref[pl.ds(i * ch, ch), :] = acc + btile


def kernel(x, W, b):
    B, K = x.shape
    N = W.shape[0]
    R = B // PACK
    Kp = K * PACK
    x2 = x.reshape(R, Kp)
    b2 = b.reshape(1, N)
    yp = pl.pallas_call(
        _linear_kernel,
        in_specs=[
            pl.BlockSpec(memory_space=pltpu.MemorySpace.HBM),
            pl.BlockSpec((N, K), lambda: (0, 0)),
            pl.BlockSpec((1, N), lambda: (0, 0)),
        ],
        out_specs=pl.BlockSpec((R, K), lambda: (0, 0)),
        out_shape=jax.ShapeDtypeStruct((R, K), x.dtype),
        scratch_shapes=[
            pltpu.VMEM((R, Kp), jnp.float32),
            pltpu.VMEM((K, Kp), jnp.float32),
            pltpu.SemaphoreType.DMA((NCHUNK,)),
        ],
    )(x2, W, b2)
    return yp.reshape(B, N)


# speedup vs baseline: 1.0308x; 1.0308x over previous
import jax
import jax.numpy as jnp
from jax.experimental import pallas as pl
from jax.experimental.pallas import tpu as pltpu

NDMA = 16
NCMP = 4
PACK = 8


def _linear_kernel(x_hbm, w_ref, b_ref, o_ref, xbuf, sems):
    R = x_hbm.shape[0]
    Kp = x_hbm.shape[1]
    N = w_ref.shape[0]
    K = w_ref.shape[1]
    chd = R // NDMA
    chc = R // NCMP

    for i in range(NDMA):
        pltpu.make_async_copy(
            x_hbm.at[pl.ds(i * chd, chd), :],
            xbuf.at[pl.ds(i * chd, chd), :],
            sems.at[i],
        ).start()

    wt = jnp.transpose(w_ref[...])
    wtile = jnp.tile(wt, (PACK, PACK))
    rows = jax.lax.broadcasted_iota(jnp.int32, (Kp, K), 0)
    cols = jax.lax.broadcasted_iota(jnp.int32, (Kp, K), 1)
    wbig = jnp.where((rows // K) == (cols // N), wtile, 0.0)

    btile = jnp.concatenate([b_ref[...]] * PACK, axis=1)

    per = NDMA // NCMP
    for i in range(NCMP):
        for s in range(per):
            pltpu.make_async_copy(
                x_hbm.at[pl.ds((i * per + s) * chd, chd), :],
                xbuf.at[pl.ds((i * per + s) * chd, chd), :],
                sems.at[i * per + s],
            ).wait()
        acc = jax.lax.dot_general(
            xbuf[pl.ds(i * chc, chc), :], wbig,
            dimension_numbers=(((1,), (0,)), ((), ())),
            preferred_element_type=jnp.float32,
        )
        o_ref[pl.ds(i * chc, chc), :] = acc + btile


def kernel(x, W, b):
    B, K = x.shape
    N = W.shape[0]
    R = B // PACK
    Kp = K * PACK
    x2 = x.reshape(R, Kp)
    b2 = b.reshape(1, N)
    yp = pl.pallas_call(
        _linear_kernel,
        in_specs=[
            pl.BlockSpec(memory_space=pltpu.MemorySpace.HBM),
            pl.BlockSpec((N, K), lambda: (0, 0)),
            pl.BlockSpec((1, N), lambda: (0, 0)),
        ],
        out_specs=pl.BlockSpec((R, K), lambda: (0, 0)),
        out_shape=jax.ShapeDtypeStruct((R, K), x.dtype),
        scratch_shapes=[
            pltpu.VMEM((R, Kp), jnp.float32),
            pltpu.SemaphoreType.DMA((NDMA,)),
        ],
    )(x2, W, b2)
    return yp.reshape(B, N)


# speedup vs baseline: 3.5120x; 3.4070x over previous
import jax
import jax.numpy as jnp
from jax.experimental import pallas as pl
from jax.experimental.pallas import tpu as pltpu

NCHUNK = 16


def _linear_kernel(x_hbm, w_ref, b_ref, o_ref, xbuf, sems):
    B = x_hbm.shape[0]
    ch = B // NCHUNK

    for i in range(NCHUNK):
        pltpu.make_async_copy(
            x_hbm.at[pl.ds(i * ch, ch), :],
            xbuf.at[pl.ds(i * ch, ch), :],
            sems.at[i],
        ).start()

    for i in range(NCHUNK):
        pltpu.make_async_copy(
            x_hbm.at[pl.ds(i * ch, ch), :],
            xbuf.at[pl.ds(i * ch, ch), :],
            sems.at[i],
        ).wait()
        acc = jax.lax.dot_general(
            xbuf[pl.ds(i * ch, ch), :], w_ref[...],
            dimension_numbers=(((1,), (1,)), ((), ())),
            preferred_element_type=jnp.float32,
        )
        o_ref[:, pl.ds(i * ch, ch)] = jnp.transpose(acc + b_ref[...])


def kernel(x, W, b):
    B, K = x.shape
    N = W.shape[0]
    b2 = b.reshape(1, N)
    yt = pl.pallas_call(
        _linear_kernel,
        in_specs=[
            pl.BlockSpec(memory_space=pltpu.MemorySpace.HBM),
            pl.BlockSpec((N, K), lambda: (0, 0)),
            pl.BlockSpec((1, N), lambda: (0, 0)),
        ],
        out_specs=pl.BlockSpec((N, B), lambda: (0, 0)),
        out_shape=jax.ShapeDtypeStruct((N, B), x.dtype),
        scratch_shapes=[
            pltpu.VMEM((B, K), jnp.float32),
            pltpu.SemaphoreType.DMA((NCHUNK,)),
        ],
    )(x, W, b2)
    return yt.T


# speedup vs baseline: 3.5413x; 1.0083x over previous
import jax
import jax.numpy as jnp
from jax.experimental import pallas as pl
from jax.experimental.pallas import tpu as pltpu

NCHUNK = 16


def _linear_kernel(x_hbm, w_ref, b_ref, o_ref, xbuf, sems):
    B = x_hbm.shape[0]
    ch = B // NCHUNK

    for i in range(NCHUNK):
        pltpu.make_async_copy(
            x_hbm.at[pl.ds(i * ch, ch), :],
            xbuf.at[pl.ds(i * ch, ch), :],
            sems.at[i],
        ).start()

    b_col = jnp.transpose(b_ref[...])
    for i in range(NCHUNK):
        pltpu.make_async_copy(
            x_hbm.at[pl.ds(i * ch, ch), :],
            xbuf.at[pl.ds(i * ch, ch), :],
            sems.at[i],
        ).wait()
        acc = jax.lax.dot_general(
            xbuf[pl.ds(i * ch, ch), :], w_ref[...],
            dimension_numbers=(((1,), (1,)), ((), ())),
            preferred_element_type=jnp.float32,
        )
        o_ref[:, pl.ds(i * ch, ch)] = jnp.transpose(acc) + b_col


def kernel(x, W, b):
    B, K = x.shape
    N = W.shape[0]
    b2 = b.reshape(1, N)
    yt = pl.pallas_call(
        _linear_kernel,
        in_specs=[
            pl.BlockSpec(memory_space=pltpu.MemorySpace.HBM),
            pl.BlockSpec((N, K), lambda: (0, 0)),
            pl.BlockSpec((1, N), lambda: (0, 0)),
        ],
        out_specs=pl.BlockSpec((N, B), lambda: (0, 0)),
        out_shape=jax.ShapeDtypeStruct((N, B), x.dtype),
        scratch_shapes=[
            pltpu.VMEM((B, K), jnp.float32),
            pltpu.SemaphoreType.DMA((NCHUNK,)),
        ],
    )(x, W, b2)
    return yt.T


# speedup vs baseline: 4.1085x; 1.1602x over previous
import jax
import jax.numpy as jnp
from jax.experimental import pallas as pl
from jax.experimental.pallas import tpu as pltpu

NDMA = 16
NCMP = 8
DEPTH = 16


def _linear_kernel(x_hbm, w_ref, b_ref, o_ref, xbuf, sems):
    B = x_hbm.shape[0]
    chd = B // NDMA
    chc = B // NCMP

    def start(i):
        pltpu.make_async_copy(
            x_hbm.at[pl.ds(i * chd, chd), :],
            xbuf.at[pl.ds(i * chd, chd), :],
            sems.at[i],
        ).start()

    for i in range(DEPTH):
        start(i)

    b_col = jnp.transpose(b_ref[...])
    per = NDMA // NCMP
    for c in range(NCMP):
        for s in range(c * per, (c + 1) * per):
            pltpu.make_async_copy(
                x_hbm.at[pl.ds(s * chd, chd), :],
                xbuf.at[pl.ds(s * chd, chd), :],
                sems.at[s],
            ).wait()
            if s + DEPTH < NDMA:
                start(s + DEPTH)
        acc = jax.lax.dot_general(
            xbuf[pl.ds(c * chc, chc), :], w_ref[...],
            dimension_numbers=(((1,), (1,)), ((), ())),
            preferred_element_type=jnp.float32,
        )
        o_ref[:, pl.ds(c * chc, chc)] = jnp.transpose(acc) + b_col


def kernel(x, W, b):
    B, K = x.shape
    N = W.shape[0]
    b2 = b.reshape(1, N)
    yt = pl.pallas_call(
        _linear_kernel,
        in_specs=[
            pl.BlockSpec(memory_space=pltpu.MemorySpace.HBM),
            pl.BlockSpec((N, K), lambda: (0, 0)),
            pl.BlockSpec((1, N), lambda: (0, 0)),
        ],
        out_specs=pl.BlockSpec((N, B), lambda: (0, 0)),
        out_shape=jax.ShapeDtypeStruct((N, B), x.dtype),
        scratch_shapes=[
            pltpu.VMEM((B, K), jnp.float32),
            pltpu.SemaphoreType.DMA((NDMA,)),
        ],
    )(x, W, b2)
    return yt.T


# speedup vs baseline: 4.4005x; 1.0711x over previous
import jax
import jax.numpy as jnp
from jax.experimental import pallas as pl
from jax.experimental.pallas import tpu as pltpu

NDMA = 16
NCMP = 4
DEPTH = 16


def _linear_kernel(x_hbm, w_ref, b_ref, o_ref, xbuf, sems):
    B = x_hbm.shape[0]
    chd = B // NDMA
    chc = B // NCMP

    def start(i):
        pltpu.make_async_copy(
            x_hbm.at[pl.ds(i * chd, chd), :],
            xbuf.at[pl.ds(i * chd, chd), :],
            sems.at[i],
        ).start()

    for i in range(DEPTH):
        start(i)

    b_col = jnp.transpose(b_ref[...])
    per = NDMA // NCMP
    for c in range(NCMP):
        for s in range(c * per, (c + 1) * per):
            pltpu.make_async_copy(
                x_hbm.at[pl.ds(s * chd, chd), :],
                xbuf.at[pl.ds(s * chd, chd), :],
                sems.at[s],
            ).wait()
            if s + DEPTH < NDMA:
                start(s + DEPTH)
        acc = jax.lax.dot_general(
            xbuf[pl.ds(c * chc, chc), :], w_ref[...],
            dimension_numbers=(((1,), (1,)), ((), ())),
            preferred_element_type=jnp.float32,
        )
        o_ref[:, pl.ds(c * chc, chc)] = jnp.transpose(acc) + b_col


def kernel(x, W, b):
    B, K = x.shape
    N = W.shape[0]
    b2 = b.reshape(1, N)
    yt = pl.pallas_call(
        _linear_kernel,
        in_specs=[
            pl.BlockSpec(memory_space=pltpu.MemorySpace.HBM),
            pl.BlockSpec((N, K), lambda: (0, 0)),
            pl.BlockSpec((1, N), lambda: (0, 0)),
        ],
        out_specs=pl.BlockSpec((N, B), lambda: (0, 0)),
        out_shape=jax.ShapeDtypeStruct((N, B), x.dtype),
        scratch_shapes=[
            pltpu.VMEM((B, K), jnp.float32),
            pltpu.SemaphoreType.DMA((NDMA,)),
        ],
    )(x, W, b2)
    return yt.T
